# baseline (speedup 1.0000x reference)
import jax
import jax.numpy as jnp
from jax import lax
from jax.experimental import pallas as pl
from jax.experimental.pallas import tpu as pltpu
from jax.experimental.pallas import tpu_sc as plsc

N = 10000
N_PAD = 10240
E = 320000
D = 128
H = 64
HP = 128
LAYERS = 6
OUT = 64
NP2 = N_PAD // 2

NC = 2
NS = 16
NW = NC * NS
CH = 128
NCHUNK = 80
EPW = NCHUNK * CH
E_PAD = NW * EPW
NBUF = 8
GAHEAD = 6
RPT = N_PAD // NS

ROWS_BLK = 1280
GRID = NP2 // ROWS_BLK


def _sc_agg_body(p_hbm, src_hbm, dst_hbm, zeros_hbm, out_hbm,
                 srcs_v, dsts_v, bufs_v, agg_sh, gsem, ssem):
    cid = lax.axis_index("c")
    sid = lax.axis_index("s")
    wid = cid * NS + sid
    c0 = pltpu.async_copy(zeros_hbm, agg_sh.at[pl.ds(sid * RPT, RPT)],
                          gsem.at[0])
    c1 = pltpu.async_copy(src_hbm.at[wid], srcs_v, gsem.at[1])
    c2 = pltpu.async_copy(dst_hbm.at[wid], dsts_v, gsem.at[2])
    c0.wait()
    c1.wait()
    c2.wait()
    plsc.subcore_barrier()

    def fire_gather(j, b):
        pltpu.async_copy(p_hbm.at[srcs_v.at[j]], bufs_v.at[b], gsem.at[b])

    def wait_gather(j, b):
        pltpu.make_async_copy(p_hbm.at[srcs_v.at[j]], bufs_v.at[b],
                              gsem.at[b]).wait()

    def fire_scatter(j, b):
        pltpu.async_copy(bufs_v.at[b], agg_sh.at[dsts_v.at[j]], ssem.at[b],
                         add=True)

    def wait_scatter(j, b):
        pltpu.make_async_copy(bufs_v.at[b], agg_sh.at[dsts_v.at[j]],
                              ssem.at[b]).wait()

    for b in range(GAHEAD):
        fire_gather(b, b)

    def outer(jj, carry):
        for b in range(NBUF):
            j = jj * NBUF + b
            bg = (b + GAHEAD) % NBUF

            @pl.when(j >= 2)
            def _():
                wait_scatter(j - 2, bg)

            @pl.when(j + GAHEAD < NCHUNK)
            def _():
                fire_gather(j + GAHEAD, bg)

            wait_gather(j, b)
            fire_scatter(j, b)
        return carry

    lax.fori_loop(0, NCHUNK // NBUF, outer, 0)
    wait_scatter(NCHUNK - 2, (NCHUNK - 2) % NBUF)
    wait_scatter(NCHUNK - 1, (NCHUNK - 1) % NBUF)
    plsc.subcore_barrier()
    pltpu.sync_copy(agg_sh.at[pl.ds(sid * RPT, RPT)],
                    out_hbm.at[cid, pl.ds(sid * RPT, RPT)])


_sc_agg = pl.kernel(
    _sc_agg_body,
    mesh=plsc.VectorSubcoreMesh(core_axis_name="c", subcore_axis_name="s"),
    compiler_params=pltpu.CompilerParams(use_tc_tiling_on_sc=False),
    out_type=jax.ShapeDtypeStruct((NC, N_PAD, H), jnp.float32),
    scratch_types=[
        pltpu.VMEM((NCHUNK, CH), jnp.int32),
        pltpu.VMEM((NCHUNK, CH), jnp.int32),
        pltpu.VMEM((NBUF, CH, H), jnp.float32),
        pltpu.VMEM_SHARED((N_PAD, H), jnp.float32),
        pltpu.SemaphoreType.DMA((NBUF,)),
        pltpu.SemaphoreType.DMA((NBUF,)),
    ],
)


def _tc0_body(x_ref, w_ref, ws_ref, bias_ref, p_ref, s_ref):
    xb = x_ref[...]
    p_ref[...] = jnp.dot(xb, w_ref[...], preferred_element_type=jnp.float32)
    s_ref[...] = (jnp.dot(xb, ws_ref[...], preferred_element_type=jnp.float32)
                  + bias_ref[...])


_tc0 = pl.pallas_call(
    _tc0_body,
    grid=(GRID,),
    in_specs=[
        pl.BlockSpec((ROWS_BLK, 2 * D), lambda i: (i, 0)),
        pl.BlockSpec((2 * D, HP), lambda i: (0, 0)),
        pl.BlockSpec((2 * D, HP), lambda i: (0, 0)),
        pl.BlockSpec((1, HP), lambda i: (0, 0)),
    ],
    out_specs=[
        pl.BlockSpec((ROWS_BLK, HP), lambda i: (i, 0)),
        pl.BlockSpec((ROWS_BLK, HP), lambda i: (i, 0)),
    ],
    out_shape=[
        jax.ShapeDtypeStruct((NP2, HP), jnp.float32),
        jax.ShapeDtypeStruct((NP2, HP), jnp.float32),
    ],
)


def _tca_body(a0_ref, a1_ref, s_ref, w_ref, h_out, p_out):
    h = jnp.maximum(a0_ref[0] + a1_ref[0] + s_ref[...], 0.0)
    h_out[...] = h
    p_out[...] = jnp.dot(h, w_ref[...], preferred_element_type=jnp.float32)


_tca = pl.pallas_call(
    _tca_body,
    grid=(GRID,),
    in_specs=[
        pl.BlockSpec((1, ROWS_BLK, HP), lambda i: (0, i, 0)),
        pl.BlockSpec((1, ROWS_BLK, HP), lambda i: (1, i, 0)),
        pl.BlockSpec((ROWS_BLK, HP), lambda i: (i, 0)),
        pl.BlockSpec((HP, HP), lambda i: (0, 0)),
    ],
    out_specs=[
        pl.BlockSpec((ROWS_BLK, HP), lambda i: (i, 0)),
        pl.BlockSpec((ROWS_BLK, HP), lambda i: (i, 0)),
    ],
    out_shape=[
        jax.ShapeDtypeStruct((NP2, HP), jnp.float32),
        jax.ShapeDtypeStruct((NP2, HP), jnp.float32),
    ],
)


def _tcb_body(h_ref, ws_ref, bias_ref, wl_ref, acc_ref, s_out, acc_out):
    h = h_ref[...]
    s_out[...] = (jnp.dot(h, ws_ref[...], preferred_element_type=jnp.float32)
                  + bias_ref[...])
    acc_out[...] = acc_ref[...] + jnp.dot(
        h, wl_ref[...], preferred_element_type=jnp.float32)


_tcb = pl.pallas_call(
    _tcb_body,
    grid=(GRID,),
    in_specs=[
        pl.BlockSpec((ROWS_BLK, HP), lambda i: (i, 0)),
        pl.BlockSpec((HP, HP), lambda i: (0, 0)),
        pl.BlockSpec((1, HP), lambda i: (0, 0)),
        pl.BlockSpec((HP, 2 * OUT), lambda i: (0, 0)),
        pl.BlockSpec((ROWS_BLK, 2 * OUT), lambda i: (i, 0)),
    ],
    out_specs=[
        pl.BlockSpec((ROWS_BLK, HP), lambda i: (i, 0)),
        pl.BlockSpec((ROWS_BLK, 2 * OUT), lambda i: (i, 0)),
    ],
    out_shape=[
        jax.ShapeDtypeStruct((NP2, HP), jnp.float32),
        jax.ShapeDtypeStruct((NP2, 2 * OUT), jnp.float32),
    ],
)


def _tcfin_body(a0_ref, a1_ref, s_ref, wl_ref, bl_ref, acc_ref, out_ref):
    h = jnp.maximum(a0_ref[0] + a1_ref[0] + s_ref[...], 0.0)
    out_ref[...] = (acc_ref[...] + bl_ref[...]
                    + jnp.dot(h, wl_ref[...],
                              preferred_element_type=jnp.float32))


_tcfin = pl.pallas_call(
    _tcfin_body,
    grid=(GRID,),
    in_specs=[
        pl.BlockSpec((1, ROWS_BLK, HP), lambda i: (0, i, 0)),
        pl.BlockSpec((1, ROWS_BLK, HP), lambda i: (1, i, 0)),
        pl.BlockSpec((ROWS_BLK, HP), lambda i: (i, 0)),
        pl.BlockSpec((HP, 2 * OUT), lambda i: (0, 0)),
        pl.BlockSpec((1, 2 * OUT), lambda i: (0, 0)),
        pl.BlockSpec((ROWS_BLK, 2 * OUT), lambda i: (i, 0)),
    ],
    out_specs=pl.BlockSpec((ROWS_BLK, 2 * OUT), lambda i: (i, 0)),
    out_shape=jax.ShapeDtypeStruct((NP2, 2 * OUT), jnp.float32),
)


def _bdiag(w):
    z = jnp.zeros_like(w)
    return jnp.concatenate(
        [jnp.concatenate([w, z], axis=1), jnp.concatenate([z, w], axis=1)],
        axis=0)


def kernel(x, edge_index, W0, b0, Ws0, bs0, bb0, W, b, Ws, bs, bb, Wl, bl):
    n_extra = E_PAD - E
    pad_src = (jnp.arange(n_extra, dtype=jnp.int32) * 37) % N
    pad_dst = N + (jnp.arange(n_extra, dtype=jnp.int32) % (N_PAD - N))
    src = jnp.concatenate([edge_index[0], pad_src]).reshape(NW, NCHUNK, CH)
    dst = jnp.concatenate([edge_index[1], pad_dst]).reshape(NW, NCHUNK, CH)
    zeros_nh = jnp.zeros((RPT, H), jnp.float32)
    x2 = jnp.pad(x, ((0, N_PAD - N), (0, 0))).reshape(NP2, 2 * D)

    bias0 = jnp.tile((b0 + bs0 + bb0).reshape(1, H), (1, 2))
    p2, s2 = _tc0(x2, _bdiag(W0), _bdiag(Ws0), bias0)
    acc2 = jnp.zeros((NP2, 2 * OUT), jnp.float32)
    for i in range(LAYERS - 1):
        agg = _sc_agg(p2.reshape(N_PAD, H), src, dst, zeros_nh)
        agg2 = agg.reshape(NC, NP2, HP)
        bias_i = jnp.tile((b[i] + bs[i] + bb[i]).reshape(1, H), (1, 2))
        h2, p2 = _tca(agg2, agg2, s2, _bdiag(W[i]))
        s2, acc2 = _tcb(h2, _bdiag(Ws[i]), bias_i,
                        _bdiag(Wl[i * H:(i + 1) * H]), acc2)
    agg = _sc_agg(p2.reshape(N_PAD, H), src, dst, zeros_nh)
    agg2 = agg.reshape(NC, NP2, HP)
    bl2 = jnp.tile(bl.reshape(1, OUT), (1, 2))
    out2 = _tcfin(agg2, agg2, s2, _bdiag(Wl[(LAYERS - 1) * H:]), bl2, acc2)
    return out2.reshape(N_PAD, OUT)[:N]


# speedup vs baseline: 1.0168x; 1.0168x over previous
import jax
import jax.numpy as jnp
from jax import lax
from jax.experimental import pallas as pl
from jax.experimental.pallas import tpu as pltpu
from jax.experimental.pallas import tpu_sc as plsc

N = 10000
N_PAD = 10240
E = 320000
D = 128
H = 64
HP = 128
LAYERS = 6
OUT = 64
NP2 = N_PAD // 2

NC = 2
NS = 16
NW = NC * NS
CH = 128
NCHUNK = 80
EPW = NCHUNK * CH
E_PAD = NW * EPW
NBUF = 8
GAHEAD = 6
RPT = N_PAD // NS

ROWS_BLK = 2560
GRID = NP2 // ROWS_BLK


def _sc_agg_body(p_hbm, src_hbm, dst_hbm, zeros_hbm, out_hbm,
                 srcs_v, dsts_v, bufs_v, agg_sh, gsem, ssem):
    cid = lax.axis_index("c")
    sid = lax.axis_index("s")
    wid = cid * NS + sid
    c0 = pltpu.async_copy(zeros_hbm, agg_sh.at[pl.ds(sid * RPT, RPT)],
                          gsem.at[0])
    c1 = pltpu.async_copy(src_hbm.at[wid], srcs_v, gsem.at[1])
    c2 = pltpu.async_copy(dst_hbm.at[wid], dsts_v, gsem.at[2])
    c0.wait()
    c1.wait()
    c2.wait()
    plsc.subcore_barrier()

    def fire_gather(j, b):
        pltpu.async_copy(p_hbm.at[srcs_v.at[j]], bufs_v.at[b], gsem.at[b])

    def wait_gather(j, b):
        pltpu.make_async_copy(p_hbm.at[srcs_v.at[j]], bufs_v.at[b],
                              gsem.at[b]).wait()

    def fire_scatter(j, b):
        pltpu.async_copy(bufs_v.at[b], agg_sh.at[dsts_v.at[j]], ssem.at[b],
                         add=True)

    def wait_scatter(j, b):
        pltpu.make_async_copy(bufs_v.at[b], agg_sh.at[dsts_v.at[j]],
                              ssem.at[b]).wait()

    for b in range(GAHEAD):
        fire_gather(b, b)

    def outer(jj, carry):
        for b in range(NBUF):
            j = jj * NBUF + b
            bg = (b + GAHEAD) % NBUF

            @pl.when(j >= 2)
            def _():
                wait_scatter(j - 2, bg)

            @pl.when(j + GAHEAD < NCHUNK)
            def _():
                fire_gather(j + GAHEAD, bg)

            wait_gather(j, b)
            fire_scatter(j, b)
        return carry

    lax.fori_loop(0, NCHUNK // NBUF, outer, 0)
    wait_scatter(NCHUNK - 2, (NCHUNK - 2) % NBUF)
    wait_scatter(NCHUNK - 1, (NCHUNK - 1) % NBUF)
    plsc.subcore_barrier()
    pltpu.sync_copy(agg_sh.at[pl.ds(sid * RPT, RPT)],
                    out_hbm.at[cid, pl.ds(sid * RPT, RPT)])


_sc_agg = pl.kernel(
    _sc_agg_body,
    mesh=plsc.VectorSubcoreMesh(core_axis_name="c", subcore_axis_name="s"),
    compiler_params=pltpu.CompilerParams(use_tc_tiling_on_sc=False),
    out_type=jax.ShapeDtypeStruct((NC, N_PAD, H), jnp.float32),
    scratch_types=[
        pltpu.VMEM((NCHUNK, CH), jnp.int32),
        pltpu.VMEM((NCHUNK, CH), jnp.int32),
        pltpu.VMEM((NBUF, CH, H), jnp.float32),
        pltpu.VMEM_SHARED((N_PAD, H), jnp.float32),
        pltpu.SemaphoreType.DMA((NBUF,)),
        pltpu.SemaphoreType.DMA((NBUF,)),
    ],
)


def _tc0_body(x_ref, w_ref, ws_ref, bias_ref, p_ref, s_ref):
    xb = x_ref[...]
    p_ref[...] = jnp.dot(xb, w_ref[...], preferred_element_type=jnp.float32)
    s_ref[...] = (jnp.dot(xb, ws_ref[...], preferred_element_type=jnp.float32)
                  + bias_ref[...])


_tc0 = pl.pallas_call(
    _tc0_body,
    grid=(GRID,),
    in_specs=[
        pl.BlockSpec((ROWS_BLK, 2 * D), lambda i: (i, 0)),
        pl.BlockSpec((2 * D, HP), lambda i: (0, 0)),
        pl.BlockSpec((2 * D, HP), lambda i: (0, 0)),
        pl.BlockSpec((1, HP), lambda i: (0, 0)),
    ],
    out_specs=[
        pl.BlockSpec((ROWS_BLK, HP), lambda i: (i, 0)),
        pl.BlockSpec((ROWS_BLK, HP), lambda i: (i, 0)),
    ],
    out_shape=[
        jax.ShapeDtypeStruct((NP2, HP), jnp.float32),
        jax.ShapeDtypeStruct((NP2, HP), jnp.float32),
    ],
)


def _tca_body(a0_ref, a1_ref, s_ref, w_ref, h_out, p_out):
    h = jnp.maximum(a0_ref[0] + a1_ref[0] + s_ref[...], 0.0)
    h_out[...] = h
    p_out[...] = jnp.dot(h, w_ref[...], preferred_element_type=jnp.float32)


_tca = pl.pallas_call(
    _tca_body,
    grid=(GRID,),
    in_specs=[
        pl.BlockSpec((1, ROWS_BLK, HP), lambda i: (0, i, 0)),
        pl.BlockSpec((1, ROWS_BLK, HP), lambda i: (1, i, 0)),
        pl.BlockSpec((ROWS_BLK, HP), lambda i: (i, 0)),
        pl.BlockSpec((HP, HP), lambda i: (0, 0)),
    ],
    out_specs=[
        pl.BlockSpec((ROWS_BLK, HP), lambda i: (i, 0)),
        pl.BlockSpec((ROWS_BLK, HP), lambda i: (i, 0)),
    ],
    out_shape=[
        jax.ShapeDtypeStruct((NP2, HP), jnp.float32),
        jax.ShapeDtypeStruct((NP2, HP), jnp.float32),
    ],
)


def _tcb_body(h_ref, ws_ref, bias_ref, wl_ref, acc_ref, s_out, acc_out):
    h = h_ref[...]
    s_out[...] = (jnp.dot(h, ws_ref[...], preferred_element_type=jnp.float32)
                  + bias_ref[...])
    acc_out[...] = acc_ref[...] + jnp.dot(
        h, wl_ref[...], preferred_element_type=jnp.float32)


_tcb = pl.pallas_call(
    _tcb_body,
    grid=(GRID,),
    in_specs=[
        pl.BlockSpec((ROWS_BLK, HP), lambda i: (i, 0)),
        pl.BlockSpec((HP, HP), lambda i: (0, 0)),
        pl.BlockSpec((1, HP), lambda i: (0, 0)),
        pl.BlockSpec((HP, 2 * OUT), lambda i: (0, 0)),
        pl.BlockSpec((ROWS_BLK, 2 * OUT), lambda i: (i, 0)),
    ],
    out_specs=[
        pl.BlockSpec((ROWS_BLK, HP), lambda i: (i, 0)),
        pl.BlockSpec((ROWS_BLK, 2 * OUT), lambda i: (i, 0)),
    ],
    out_shape=[
        jax.ShapeDtypeStruct((NP2, HP), jnp.float32),
        jax.ShapeDtypeStruct((NP2, 2 * OUT), jnp.float32),
    ],
)


def _tcfin_body(a0_ref, a1_ref, s_ref, wl_ref, bl_ref, acc_ref, out_ref):
    h = jnp.maximum(a0_ref[0] + a1_ref[0] + s_ref[...], 0.0)
    out_ref[...] = (acc_ref[...] + bl_ref[...]
                    + jnp.dot(h, wl_ref[...],
                              preferred_element_type=jnp.float32))


_tcfin = pl.pallas_call(
    _tcfin_body,
    grid=(GRID,),
    in_specs=[
        pl.BlockSpec((1, ROWS_BLK, HP), lambda i: (0, i, 0)),
        pl.BlockSpec((1, ROWS_BLK, HP), lambda i: (1, i, 0)),
        pl.BlockSpec((ROWS_BLK, HP), lambda i: (i, 0)),
        pl.BlockSpec((HP, 2 * OUT), lambda i: (0, 0)),
        pl.BlockSpec((1, 2 * OUT), lambda i: (0, 0)),
        pl.BlockSpec((ROWS_BLK, 2 * OUT), lambda i: (i, 0)),
    ],
    out_specs=pl.BlockSpec((ROWS_BLK, 2 * OUT), lambda i: (i, 0)),
    out_shape=jax.ShapeDtypeStruct((NP2, 2 * OUT), jnp.float32),
)


def _bdiag(w):
    z = jnp.zeros_like(w)
    return jnp.concatenate(
        [jnp.concatenate([w, z], axis=1), jnp.concatenate([z, w], axis=1)],
        axis=0)


def kernel(x, edge_index, W0, b0, Ws0, bs0, bb0, W, b, Ws, bs, bb, Wl, bl):
    n_extra = E_PAD - E
    pad_src = (jnp.arange(n_extra, dtype=jnp.int32) * 37) % N
    pad_dst = N + (jnp.arange(n_extra, dtype=jnp.int32) % (N_PAD - N))
    src = jnp.concatenate([edge_index[0], pad_src]).reshape(NW, NCHUNK, CH)
    dst = jnp.concatenate([edge_index[1], pad_dst]).reshape(NW, NCHUNK, CH)
    zeros_nh = jnp.zeros((RPT, H), jnp.float32)
    x2 = jnp.pad(x, ((0, N_PAD - N), (0, 0))).reshape(NP2, 2 * D)

    bias0 = jnp.tile((b0 + bs0 + bb0).reshape(1, H), (1, 2))
    p2, s2 = _tc0(x2, _bdiag(W0), _bdiag(Ws0), bias0)
    acc2 = jnp.zeros((NP2, 2 * OUT), jnp.float32)
    for i in range(LAYERS - 1):
        agg = _sc_agg(p2.reshape(N_PAD, H), src, dst, zeros_nh)
        agg2 = agg.reshape(NC, NP2, HP)
        bias_i = jnp.tile((b[i] + bs[i] + bb[i]).reshape(1, H), (1, 2))
        h2, p2 = _tca(agg2, agg2, s2, _bdiag(W[i]))
        s2, acc2 = _tcb(h2, _bdiag(Ws[i]), bias_i,
                        _bdiag(Wl[i * H:(i + 1) * H]), acc2)
    agg = _sc_agg(p2.reshape(N_PAD, H), src, dst, zeros_nh)
    agg2 = agg.reshape(NC, NP2, HP)
    bl2 = jnp.tile(bl.reshape(1, OUT), (1, 2))
    out2 = _tcfin(agg2, agg2, s2, _bdiag(Wl[(LAYERS - 1) * H:]), bl2, acc2)
    return out2.reshape(N_PAD, OUT)[:N]
